# baseline (speedup 1.0000x reference)
import functools

import jax
import jax.numpy as jnp
from jax import lax
from jax.experimental import pallas as pl
from jax.experimental.pallas import tpu as pltpu
from jax.experimental.pallas import tpu_sc as plsc

N_NODES = 100000
N_EDGES = 1600000
B = 4096
K = 20
D_NODE = 128
D_EDGE = 16
D_TIME = 128
D_EMB = 128
H = 2
DH = D_EMB // H

NC = 2
NS = 16
NW = NC * NS
RPW = (B * K) // NW
CH = 128
NCHUNK = RPW // CH
SPW = B // NW
DEPTH = 4

BQ = 256
NB = B // BQ
BKR = BQ * K

NEG = -1e10

CB = 5000
NCB = N_NODES // CB


def _combine_body(nf, mem, out):
    out[...] = nf[...] + mem[...]


def _combine(node_features, memory):
    return pl.pallas_call(
        _combine_body,
        grid=(NCB,),
        in_specs=[pl.BlockSpec((CB, D_NODE), lambda i: (i, 0)),
                  pl.BlockSpec((CB, D_NODE), lambda i: (i, 0))],
        out_specs=pl.BlockSpec((CB, D_NODE), lambda i: (i, 0)),
        out_shape=jax.ShapeDtypeStruct((N_NODES, D_NODE), jnp.float32),
    )(node_features, memory)


def _sc_gather(comb, edge_features, nbr3, eidx3, sidx2):
    mesh = plsc.VectorSubcoreMesh(core_axis_name="c", subcore_axis_name="s")
    out_type = (
        jax.ShapeDtypeStruct((B * K, D_NODE), jnp.float32),
        jax.ShapeDtypeStruct((B * K, D_EDGE), jnp.float32),
        jax.ShapeDtypeStruct((B, D_NODE), jnp.float32),
    )
    scratch = [
        pltpu.VMEM((NCHUNK, CH), jnp.int32),
        pltpu.VMEM((NCHUNK, CH), jnp.int32),
        pltpu.VMEM((SPW,), jnp.int32),
        pltpu.VMEM((DEPTH, CH, D_NODE), jnp.float32),
        pltpu.VMEM((DEPTH, CH, D_EDGE), jnp.float32),
        pltpu.VMEM((SPW, D_NODE), jnp.float32),
    ] + [pltpu.SemaphoreType.DMA] * (2 * DEPTH + 2)

    @functools.partial(pl.kernel, out_type=out_type, mesh=mesh,
                       scratch_types=scratch,
                       compiler_params=pltpu.CompilerParams(
                           use_tc_tiling_on_sc=False))
    def body(comb_hbm, ef_hbm, nbr_hbm, eidx_hbm, sidx_hbm,
             nb_out, ef_out, src_out,
             idx_v, eidx_v, sidx_v, bufn, bufe, bufs, *sems):
        sg = sems[0:DEPTH]
        sw = sems[DEPTH:2 * DEPTH]
        ssrc, swsrc = sems[2 * DEPTH], sems[2 * DEPTH + 1]
        wid = lax.axis_index("c") * NS + lax.axis_index("s")
        pltpu.sync_copy(nbr_hbm.at[wid], idx_v)
        pltpu.sync_copy(eidx_hbm.at[wid], eidx_v)
        pltpu.sync_copy(sidx_hbm.at[wid], sidx_v)
        base = wid * RPW

        def fire_gathers(c):
            s = c % DEPTH
            return [
                pltpu.async_copy(comb_hbm.at[idx_v.at[c]], bufn.at[s], sg[s]),
                pltpu.async_copy(ef_hbm.at[eidx_v.at[c]], bufe.at[s], sg[s]),
            ]

        def fire_writes(c):
            s = c % DEPTH
            row = base + c * CH
            return [
                pltpu.async_copy(bufn.at[s], nb_out.at[pl.ds(row, CH)], sw[s]),
                pltpu.async_copy(bufe.at[s], ef_out.at[pl.ds(row, CH)], sw[s]),
            ]

        hsrc = pltpu.async_copy(comb_hbm.at[sidx_v], bufs, ssrc)

        wg = [None] * DEPTH
        wh = [None] * DEPTH
        for c in range(DEPTH):
            wg[c] = fire_gathers(c)
        for c in range(NCHUNK):
            s = c % DEPTH
            for h in wg[s]:
                h.wait()
            wh[s] = fire_writes(c)
            n = c + DEPTH
            if n < NCHUNK:
                for h in wh[s]:
                    h.wait()
                wg[s] = fire_gathers(n)
        hsrc.wait()
        hw = pltpu.async_copy(bufs, src_out.at[pl.ds(wid * SPW, SPW)], swsrc)
        for s in range(DEPTH):
            if wh[s] is not None:
                for h in wh[s]:
                    h.wait()
        hw.wait()

    return body(comb, edge_features, nbr3, eidx3, sidx2)


def _tc_body(nbg, efg, dcol, sbias, nbias, srcg, tw, tb,
             wq, wk, wv, wm1, wm2, out):
    f32 = jnp.float32
    bf16 = jnp.bfloat16

    def mm(a, b):
        return lax.dot_general(a.astype(bf16), b.astype(bf16),
                               (((1,), (0,)), ((), ())),
                               preferred_element_type=f32)

    def mm_nt(a, b):
        return lax.dot_general(a.astype(bf16), b.astype(bf16),
                               (((1,), (1,)), ((), ())),
                               preferred_element_type=f32)

    tww = tw[...]
    tbb = tb[...]
    neigh = nbg[...]
    etime = jnp.cos(dcol[...] * tww + tbb)
    ef = efg[...]
    wk_ = wk[...]
    wv_ = wv[...]
    kmat = (mm(neigh, wk_[0:D_NODE])
            + mm(etime, wk_[D_NODE:D_NODE + D_TIME])
            + mm(ef, wk_[D_NODE + D_TIME:D_NODE + D_TIME + D_EDGE]))
    vmat = (mm(neigh, wv_[0:D_NODE])
            + mm(etime, wv_[D_NODE:D_NODE + D_TIME])
            + mm(ef, wv_[D_NODE + D_TIME:D_NODE + D_TIME + D_EDGE]))

    src = srcg[...]
    wq_ = wq[...]
    stime = jnp.cos(tbb)
    q = mm(src, wq_[0:D_NODE]) + mm(stime, wq_[D_NODE:D_NODE + D_TIME])

    bias = sbias[...] + nbias[0]
    scale = f32(1.0) / jnp.sqrt(f32(DH))
    lane = lax.broadcasted_iota(jnp.int32, (1, D_EMB), 1)

    outs = []
    for h in range(H):
        headmask = ((lane >= h * DH) & (lane < (h + 1) * DH)).astype(f32)
        qh = q * headmask
        sh = mm_nt(qh, kmat) * scale + bias
        mh = jnp.max(sh, axis=1, keepdims=True)
        eh = jnp.exp(sh - mh)
        ph = eh / jnp.sum(eh, axis=1, keepdims=True)
        oh = mm(ph, vmat)
        outs.append(oh[:, h * DH:(h + 1) * DH])
    o = jnp.concatenate(outs, axis=1)

    wm1_ = wm1[...]
    hm = jnp.maximum(mm(o, wm1_[0:D_EMB]) + mm(src, wm1_[D_EMB:D_EMB + D_NODE]),
                     f32(0.0))
    out[...] = mm(hm, wm2[...])


def _tc_stage(nb_rows, ef_rows, dcol, sbias, nbias, src_rows,
              tw2, tb2, wq, wk, wv, wm1, wm2):
    full = lambda shape: pl.BlockSpec(shape, lambda i: (0,) * len(shape))
    grid_spec = pl.GridSpec(
        grid=(NB,),
        in_specs=[
            pl.BlockSpec((BKR, D_NODE), lambda i: (i, 0)),
            pl.BlockSpec((BKR, D_EDGE), lambda i: (i, 0)),
            pl.BlockSpec((BKR, 1), lambda i: (i, 0)),
            full((BQ, BKR)),
            pl.BlockSpec((1, 1, BKR), lambda i: (i, 0, 0)),
            pl.BlockSpec((BQ, D_NODE), lambda i: (i, 0)),
            full((1, D_TIME)),
            full((1, D_TIME)),
            full((D_NODE + D_TIME, D_EMB)),
            full((D_NODE + D_TIME + D_EDGE, D_EMB)),
            full((D_NODE + D_TIME + D_EDGE, D_EMB)),
            full((D_EMB + D_NODE, D_EMB)),
            full((D_EMB, D_EMB)),
        ],
        out_specs=pl.BlockSpec((BQ, D_EMB), lambda i: (i, 0)),
    )
    return pl.pallas_call(
        _tc_body,
        grid_spec=grid_spec,
        out_shape=jax.ShapeDtypeStruct((B, D_EMB), jnp.float32),
    )(nb_rows, ef_rows, dcol, sbias, nbias, src_rows,
      tw2, tb2, wq, wk, wv, wm1, wm2)


def kernel(memory, source_nodes, timestamps, neighbors, edge_idxs, edge_times,
           node_features, edge_features, time_w, time_b, Wq, Wk, Wv, Wm1, Wm2):
    nbr_flat = neighbors.reshape(-1).astype(jnp.int32)
    nbr3 = nbr_flat.reshape(NW, NCHUNK, CH)
    eidx3 = edge_idxs.reshape(-1).astype(jnp.int32).reshape(NW, NCHUNK, CH)
    sidx2 = source_nodes.astype(jnp.int32).reshape(NW, SPW)

    comb = _combine(node_features, memory)
    nb_rows, ef_rows, src_rows = _sc_gather(
        comb, edge_features, nbr3, eidx3, sidx2)

    dcol = (timestamps[:, None] - edge_times).reshape(B * K, 1)
    col = lax.broadcasted_iota(jnp.int32, (BQ, BKR), 1)
    row = lax.broadcasted_iota(jnp.int32, (BQ, BKR), 0) * K
    sbias = jnp.where((col >= row) & (col < row + K), 0.0, NEG)
    sbias = sbias.astype(jnp.float32)
    nbias = jnp.where(nbr_flat == 0, NEG, 0.0).astype(jnp.float32)
    nbias = nbias.reshape(NB, 1, BKR)
    tw2 = time_w.reshape(1, D_TIME)
    tb2 = time_b.reshape(1, D_TIME)

    return _tc_stage(nb_rows, ef_rows, dcol, sbias, nbias, src_rows,
                     tw2, tb2, Wq, Wk, Wv, Wm1, Wm2)
